# baseline (speedup 1.0000x reference)
import functools

import jax
import jax.numpy as jnp
from jax.experimental import pallas as pl
from jax.experimental.pallas import tpu as pltpu

_D_MODEL = 2048
_NUM_EXPERTS = 16
_TOP_K = 2
_BALANCE_W = 0.01
_NBUF = 3


def _body(nsteps, tile, total_tokens, x_hbm, wr_ref, wg_ref, br_ref, bg_ref,
          rl_ref, idx_ref, g_ref, gv_ref, cnt_ref, imp_ref, bl_ref, il_ref,
          xbuf, wt_ref, sems, sems2):
    i = pl.program_id(0)
    E = _NUM_EXPERTS

    h = _D_MODEL // 2

    def copy_a(j):
        slot = jax.lax.rem(j, _NBUF)
        return pltpu.make_async_copy(
            x_hbm.at[pl.ds(j * tile, tile), pl.ds(0, h)],
            xbuf.at[slot, :, pl.ds(0, h)], sems.at[slot])

    def copy_b(j):
        slot = jax.lax.rem(j, _NBUF)
        return pltpu.make_async_copy(
            x_hbm.at[pl.ds(j * tile, tile), pl.ds(h, h)],
            xbuf.at[slot, :, pl.ds(h, h)], sems2.at[slot])

    @pl.when(i == 0)
    def _prologue():
        for j in range(min(_NBUF - 1, nsteps)):
            copy_a(j).start()
            copy_b(j).start()
        wt_ref[...] = jnp.concatenate(
            [wr_ref[...].T, wg_ref[...].T], axis=1)

    @pl.when(i + _NBUF - 1 < nsteps)
    def _lookahead():
        copy_a(i + _NBUF - 1).start()
        copy_b(i + _NBUF - 1).start()

    copy_a(i).wait()
    copy_b(i).wait()
    xt = xbuf[jax.lax.rem(i, _NBUF)]

    y = jnp.dot(xt, wt_ref[...],
                preferred_element_type=jnp.float32)
    yt = y.T
    rlt = yt[:E, :] + br_ref[...].T
    glt = yt[E:, :] + bg_ref[...].T
    rl_ref[...] = rlt
    t = rlt.shape[1]
    iota = jax.lax.broadcasted_iota(jnp.int32, (E, t), 0)

    m1 = jnp.max(rlt, axis=0, keepdims=True)
    i1 = jnp.min(jnp.where(rlt == m1, iota, E), axis=0, keepdims=True)
    masked = jnp.where(iota == i1, -jnp.inf, rlt)
    m2 = jnp.max(masked, axis=0, keepdims=True)
    i2 = jnp.min(jnp.where(masked == m2, iota, E), axis=0, keepdims=True)
    idx_ref[...] = jnp.concatenate([i1, i2], axis=0)

    e2 = jnp.exp(m2 - m1)
    den = 1.0 + e2
    g_ref[...] = jnp.concatenate([1.0 / den, e2 / den], axis=0)

    gm = jnp.max(glt, axis=0, keepdims=True)
    ge = jnp.exp(glt - gm)
    gvt = ge / jnp.sum(ge, axis=0, keepdims=True)
    gv_ref[...] = gvt

    hits = (iota == i1).astype(jnp.float32) + (iota == i2).astype(jnp.float32)
    cnt = jnp.sum(hits, axis=1, keepdims=True)
    imp = jnp.sum(gvt, axis=1, keepdims=True)

    @pl.when(i == 0)
    def _init():
        cnt_ref[...] = jnp.zeros_like(cnt_ref)
        imp_ref[...] = jnp.zeros_like(imp_ref)

    cnt_ref[...] += cnt
    imp_ref[...] += imp

    @pl.when(i == nsteps - 1)
    def _finalize():
        frac = cnt_ref[...] / total_tokens
        bl_ref[...] = (_BALANCE_W
                       * (E * jnp.sum(frac * frac) - 1.0)).reshape(1, 1)
        im = imp_ref[...]
        ti = jnp.sum(im)
        ifrac = jnp.where(ti > 0, im / ti, jnp.zeros_like(im))
        il_ref[...] = (_BALANCE_W
                       * jnp.sum((ifrac - 1.0 / E) ** 2)).reshape(1, 1)


def kernel(x, Wr, br, Wg, bg):
    B, S, D = x.shape
    E = _NUM_EXPERTS
    n_tok = B * S
    tile = 2048
    nsteps = n_tok // tile

    xf = x.reshape(n_tok, D)
    br2 = br.reshape(1, E)
    bg2 = bg.reshape(1, E)

    grid_spec = pltpu.PrefetchScalarGridSpec(
        num_scalar_prefetch=0,
        grid=(nsteps,),
        in_specs=[
            pl.BlockSpec(memory_space=pl.ANY),
            pl.BlockSpec((E, D), lambda i: (0, 0)),
            pl.BlockSpec((E, D), lambda i: (0, 0)),
            pl.BlockSpec((1, E), lambda i: (0, 0)),
            pl.BlockSpec((1, E), lambda i: (0, 0)),
        ],
        scratch_shapes=[
            pltpu.MemorySpace.VMEM((_NBUF, tile, D), jnp.float32),
            pltpu.MemorySpace.VMEM((D, 2 * E), jnp.float32),
            pltpu.SemaphoreType.DMA((_NBUF,)),
            pltpu.SemaphoreType.DMA((_NBUF,)),
        ],
        out_specs=[
            pl.BlockSpec((E, tile), lambda i: (0, i)),
            pl.BlockSpec((_TOP_K, tile), lambda i: (0, i)),
            pl.BlockSpec((_TOP_K, tile), lambda i: (0, i)),
            pl.BlockSpec((E, tile), lambda i: (0, i)),
            pl.BlockSpec((E, 1), lambda i: (0, 0)),
            pl.BlockSpec((E, 1), lambda i: (0, 0)),
            pl.BlockSpec((1, 1), lambda i: (0, 0)),
            pl.BlockSpec((1, 1), lambda i: (0, 0)),
        ],
    )

    out_shapes = [
        jax.ShapeDtypeStruct((E, n_tok), jnp.float32),
        jax.ShapeDtypeStruct((_TOP_K, n_tok), jnp.int32),
        jax.ShapeDtypeStruct((_TOP_K, n_tok), jnp.float32),
        jax.ShapeDtypeStruct((E, n_tok), jnp.float32),
        jax.ShapeDtypeStruct((E, 1), jnp.float32),
        jax.ShapeDtypeStruct((E, 1), jnp.float32),
        jax.ShapeDtypeStruct((1, 1), jnp.float32),
        jax.ShapeDtypeStruct((1, 1), jnp.float32),
    ]

    body = functools.partial(_body, nsteps, tile, float(n_tok))
    rl, idx, g, gv, _, _, bl, il = pl.pallas_call(
        body,
        grid_spec=grid_spec,
        out_shape=out_shapes,
    )(xf, Wr, Wg, br2, bg2)

    return (rl.T.reshape(B, S, E),
            idx.T.reshape(B, S, _TOP_K),
            g.T.reshape(B, S, _TOP_K),
            gv.T.reshape(B, S, E),
            bl[0, 0],
            il[0, 0])


# speedup vs baseline: 1.0185x; 1.0185x over previous
import functools

import jax
import jax.numpy as jnp
from jax.experimental import pallas as pl
from jax.experimental.pallas import tpu as pltpu

_D_MODEL = 2048
_NUM_EXPERTS = 16
_TOP_K = 2
_BALANCE_W = 0.01
_NBUF = 6


def _body(nsteps, tile, total_tokens, x_hbm, wr_ref, wg_ref, br_ref, bg_ref,
          rl_ref, idx_ref, g_ref, gv_ref, cnt_ref, imp_ref, bl_ref, il_ref,
          xbuf, wt_ref, sems, sems2):
    i = pl.program_id(0)
    E = _NUM_EXPERTS

    h = _D_MODEL // 2

    def copy_a(j):
        slot = jax.lax.rem(j, _NBUF)
        return pltpu.make_async_copy(
            x_hbm.at[pl.ds(j * tile, tile), pl.ds(0, h)],
            xbuf.at[slot, :, pl.ds(0, h)], sems.at[slot])

    def copy_b(j):
        slot = jax.lax.rem(j, _NBUF)
        return pltpu.make_async_copy(
            x_hbm.at[pl.ds(j * tile, tile), pl.ds(h, h)],
            xbuf.at[slot, :, pl.ds(h, h)], sems2.at[slot])

    @pl.when(i == 0)
    def _prologue():
        for j in range(min(_NBUF - 1, nsteps)):
            copy_a(j).start()
            copy_b(j).start()
        wt_ref[...] = jnp.concatenate(
            [wr_ref[...].T, wg_ref[...].T], axis=1)

    @pl.when(i + _NBUF - 1 < nsteps)
    def _lookahead():
        copy_a(i + _NBUF - 1).start()
        copy_b(i + _NBUF - 1).start()

    copy_a(i).wait()
    copy_b(i).wait()
    xt = xbuf[jax.lax.rem(i, _NBUF)]

    y = jnp.dot(xt, wt_ref[...],
                preferred_element_type=jnp.float32)
    yt = y.T
    rlt = yt[:E, :] + br_ref[...].T
    glt = yt[E:, :] + bg_ref[...].T
    rl_ref[...] = rlt
    t = rlt.shape[1]
    iota = jax.lax.broadcasted_iota(jnp.int32, (E, t), 0)

    m1 = jnp.max(rlt, axis=0, keepdims=True)
    i1 = jnp.min(jnp.where(rlt == m1, iota, E), axis=0, keepdims=True)
    masked = jnp.where(iota == i1, -jnp.inf, rlt)
    m2 = jnp.max(masked, axis=0, keepdims=True)
    i2 = jnp.min(jnp.where(masked == m2, iota, E), axis=0, keepdims=True)
    idx_ref[...] = jnp.concatenate([i1, i2], axis=0)

    e2 = jnp.exp(m2 - m1)
    den = 1.0 + e2
    g_ref[...] = jnp.concatenate([1.0 / den, e2 / den], axis=0)

    gm = jnp.max(glt, axis=0, keepdims=True)
    ge = jnp.exp(glt - gm)
    gvt = ge / jnp.sum(ge, axis=0, keepdims=True)
    gv_ref[...] = gvt

    hits = (iota == i1).astype(jnp.float32) + (iota == i2).astype(jnp.float32)
    cnt = jnp.sum(hits, axis=1, keepdims=True)
    imp = jnp.sum(gvt, axis=1, keepdims=True)

    @pl.when(i == 0)
    def _init():
        cnt_ref[...] = jnp.zeros_like(cnt_ref)
        imp_ref[...] = jnp.zeros_like(imp_ref)

    cnt_ref[...] += cnt
    imp_ref[...] += imp

    @pl.when(i == nsteps - 1)
    def _finalize():
        frac = cnt_ref[...] / total_tokens
        bl_ref[...] = (_BALANCE_W
                       * (E * jnp.sum(frac * frac) - 1.0)).reshape(1, 1)
        im = imp_ref[...]
        ti = jnp.sum(im)
        ifrac = jnp.where(ti > 0, im / ti, jnp.zeros_like(im))
        il_ref[...] = (_BALANCE_W
                       * jnp.sum((ifrac - 1.0 / E) ** 2)).reshape(1, 1)


def kernel(x, Wr, br, Wg, bg):
    B, S, D = x.shape
    E = _NUM_EXPERTS
    n_tok = B * S
    tile = 1024
    nsteps = n_tok // tile

    xf = x.reshape(n_tok, D)
    br2 = br.reshape(1, E)
    bg2 = bg.reshape(1, E)

    grid_spec = pltpu.PrefetchScalarGridSpec(
        num_scalar_prefetch=0,
        grid=(nsteps,),
        in_specs=[
            pl.BlockSpec(memory_space=pl.ANY),
            pl.BlockSpec((E, D), lambda i: (0, 0)),
            pl.BlockSpec((E, D), lambda i: (0, 0)),
            pl.BlockSpec((1, E), lambda i: (0, 0)),
            pl.BlockSpec((1, E), lambda i: (0, 0)),
        ],
        scratch_shapes=[
            pltpu.MemorySpace.VMEM((_NBUF, tile, D), jnp.float32),
            pltpu.MemorySpace.VMEM((D, 2 * E), jnp.float32),
            pltpu.SemaphoreType.DMA((_NBUF,)),
            pltpu.SemaphoreType.DMA((_NBUF,)),
        ],
        out_specs=[
            pl.BlockSpec((E, tile), lambda i: (0, i)),
            pl.BlockSpec((_TOP_K, tile), lambda i: (0, i)),
            pl.BlockSpec((_TOP_K, tile), lambda i: (0, i)),
            pl.BlockSpec((E, tile), lambda i: (0, i)),
            pl.BlockSpec((E, 1), lambda i: (0, 0)),
            pl.BlockSpec((E, 1), lambda i: (0, 0)),
            pl.BlockSpec((1, 1), lambda i: (0, 0)),
            pl.BlockSpec((1, 1), lambda i: (0, 0)),
        ],
    )

    out_shapes = [
        jax.ShapeDtypeStruct((E, n_tok), jnp.float32),
        jax.ShapeDtypeStruct((_TOP_K, n_tok), jnp.int32),
        jax.ShapeDtypeStruct((_TOP_K, n_tok), jnp.float32),
        jax.ShapeDtypeStruct((E, n_tok), jnp.float32),
        jax.ShapeDtypeStruct((E, 1), jnp.float32),
        jax.ShapeDtypeStruct((E, 1), jnp.float32),
        jax.ShapeDtypeStruct((1, 1), jnp.float32),
        jax.ShapeDtypeStruct((1, 1), jnp.float32),
    ]

    body = functools.partial(_body, nsteps, tile, float(n_tok))
    rl, idx, g, gv, _, _, bl, il = pl.pallas_call(
        body,
        grid_spec=grid_spec,
        out_shape=out_shapes,
    )(xf, Wr, Wg, br2, bg2)

    return (rl.T.reshape(B, S, E),
            idx.T.reshape(B, S, _TOP_K),
            g.T.reshape(B, S, _TOP_K),
            gv.T.reshape(B, S, E),
            bl[0, 0],
            il[0, 0])


# speedup vs baseline: 1.0259x; 1.0073x over previous
import functools

import jax
import jax.numpy as jnp
from jax.experimental import pallas as pl
from jax.experimental.pallas import tpu as pltpu

_D_MODEL = 2048
_NUM_EXPERTS = 16
_TOP_K = 2
_BALANCE_W = 0.01
_NBUF = 8


def _body(nsteps, tile, total_tokens, x_hbm, wr_ref, wg_ref, br_ref, bg_ref,
          rl_ref, idx_ref, g_ref, gv_ref, cnt_ref, imp_ref, bl_ref, il_ref,
          xbuf, wt_ref, sems, sems2):
    i = pl.program_id(0)
    E = _NUM_EXPERTS

    h = _D_MODEL // 2

    def copy_a(j):
        slot = jax.lax.rem(j, _NBUF)
        return pltpu.make_async_copy(
            x_hbm.at[pl.ds(j * tile, tile), pl.ds(0, h)],
            xbuf.at[slot, :, pl.ds(0, h)], sems.at[slot])

    def copy_b(j):
        slot = jax.lax.rem(j, _NBUF)
        return pltpu.make_async_copy(
            x_hbm.at[pl.ds(j * tile, tile), pl.ds(h, h)],
            xbuf.at[slot, :, pl.ds(h, h)], sems2.at[slot])

    @pl.when(i == 0)
    def _prologue():
        for j in range(min(_NBUF - 1, nsteps)):
            copy_a(j).start()
            copy_b(j).start()
        wt_ref[...] = jnp.concatenate(
            [wr_ref[...].T, wg_ref[...].T], axis=1)

    @pl.when(i + _NBUF - 1 < nsteps)
    def _lookahead():
        copy_a(i + _NBUF - 1).start()
        copy_b(i + _NBUF - 1).start()

    copy_a(i).wait()
    copy_b(i).wait()
    xt = xbuf[jax.lax.rem(i, _NBUF)]

    y = jnp.dot(xt, wt_ref[...],
                preferred_element_type=jnp.float32)
    yt = y.T
    rlt = yt[:E, :] + br_ref[...].T
    glt = yt[E:, :] + bg_ref[...].T
    rl_ref[...] = rlt
    t = rlt.shape[1]
    iota = jax.lax.broadcasted_iota(jnp.int32, (E, t), 0)

    m1 = jnp.max(rlt, axis=0, keepdims=True)
    i1 = jnp.min(jnp.where(rlt == m1, iota, E), axis=0, keepdims=True)
    masked = jnp.where(iota == i1, -jnp.inf, rlt)
    m2 = jnp.max(masked, axis=0, keepdims=True)
    i2 = jnp.min(jnp.where(masked == m2, iota, E), axis=0, keepdims=True)
    idx_ref[...] = jnp.concatenate([i1, i2], axis=0)

    e2 = jnp.exp(m2 - m1)
    den = 1.0 + e2
    g_ref[...] = jnp.concatenate([1.0 / den, e2 / den], axis=0)

    gm = jnp.max(glt, axis=0, keepdims=True)
    ge = jnp.exp(glt - gm)
    gvt = ge / jnp.sum(ge, axis=0, keepdims=True)
    gv_ref[...] = gvt

    hits = (iota == i1).astype(jnp.float32) + (iota == i2).astype(jnp.float32)
    cnt = jnp.sum(hits, axis=1, keepdims=True)
    imp = jnp.sum(gvt, axis=1, keepdims=True)

    @pl.when(i == 0)
    def _init():
        cnt_ref[...] = jnp.zeros_like(cnt_ref)
        imp_ref[...] = jnp.zeros_like(imp_ref)

    cnt_ref[...] += cnt
    imp_ref[...] += imp

    @pl.when(i == nsteps - 1)
    def _finalize():
        frac = cnt_ref[...] / total_tokens
        bl_ref[...] = (_BALANCE_W
                       * (E * jnp.sum(frac * frac) - 1.0)).reshape(1, 1)
        im = imp_ref[...]
        ti = jnp.sum(im)
        ifrac = jnp.where(ti > 0, im / ti, jnp.zeros_like(im))
        il_ref[...] = (_BALANCE_W
                       * jnp.sum((ifrac - 1.0 / E) ** 2)).reshape(1, 1)


def kernel(x, Wr, br, Wg, bg):
    B, S, D = x.shape
    E = _NUM_EXPERTS
    n_tok = B * S
    tile = 512
    nsteps = n_tok // tile

    xf = x.reshape(n_tok, D)
    br2 = br.reshape(1, E)
    bg2 = bg.reshape(1, E)

    grid_spec = pltpu.PrefetchScalarGridSpec(
        num_scalar_prefetch=0,
        grid=(nsteps,),
        in_specs=[
            pl.BlockSpec(memory_space=pl.ANY),
            pl.BlockSpec((E, D), lambda i: (0, 0)),
            pl.BlockSpec((E, D), lambda i: (0, 0)),
            pl.BlockSpec((1, E), lambda i: (0, 0)),
            pl.BlockSpec((1, E), lambda i: (0, 0)),
        ],
        scratch_shapes=[
            pltpu.MemorySpace.VMEM((_NBUF, tile, D), jnp.float32),
            pltpu.MemorySpace.VMEM((D, 2 * E), jnp.float32),
            pltpu.SemaphoreType.DMA((_NBUF,)),
            pltpu.SemaphoreType.DMA((_NBUF,)),
        ],
        out_specs=[
            pl.BlockSpec((E, tile), lambda i: (0, i)),
            pl.BlockSpec((_TOP_K, tile), lambda i: (0, i)),
            pl.BlockSpec((_TOP_K, tile), lambda i: (0, i)),
            pl.BlockSpec((E, tile), lambda i: (0, i)),
            pl.BlockSpec((E, 1), lambda i: (0, 0)),
            pl.BlockSpec((E, 1), lambda i: (0, 0)),
            pl.BlockSpec((1, 1), lambda i: (0, 0)),
            pl.BlockSpec((1, 1), lambda i: (0, 0)),
        ],
    )

    out_shapes = [
        jax.ShapeDtypeStruct((E, n_tok), jnp.float32),
        jax.ShapeDtypeStruct((_TOP_K, n_tok), jnp.int32),
        jax.ShapeDtypeStruct((_TOP_K, n_tok), jnp.float32),
        jax.ShapeDtypeStruct((E, n_tok), jnp.float32),
        jax.ShapeDtypeStruct((E, 1), jnp.float32),
        jax.ShapeDtypeStruct((E, 1), jnp.float32),
        jax.ShapeDtypeStruct((1, 1), jnp.float32),
        jax.ShapeDtypeStruct((1, 1), jnp.float32),
    ]

    body = functools.partial(_body, nsteps, tile, float(n_tok))
    rl, idx, g, gv, _, _, bl, il = pl.pallas_call(
        body,
        grid_spec=grid_spec,
        out_shape=out_shapes,
    )(xf, Wr, Wg, br2, bg2)

    return (rl.T.reshape(B, S, E),
            idx.T.reshape(B, S, _TOP_K),
            g.T.reshape(B, S, _TOP_K),
            gv.T.reshape(B, S, E),
            bl[0, 0],
            il[0, 0])


# speedup vs baseline: 1.0263x; 1.0004x over previous
import functools

import jax
import jax.numpy as jnp
from jax.experimental import pallas as pl
from jax.experimental.pallas import tpu as pltpu

_D_MODEL = 2048
_NUM_EXPERTS = 16
_TOP_K = 2
_BALANCE_W = 0.01
_NBUF = 12


def _body(nsteps, tile, total_tokens, x_hbm, wr_ref, wg_ref, br_ref, bg_ref,
          rl_ref, idx_ref, g_ref, gv_ref, cnt_ref, imp_ref, bl_ref, il_ref,
          xbuf, wt_ref, sems, sems2):
    i = pl.program_id(0)
    E = _NUM_EXPERTS

    h = _D_MODEL // 2

    def copy_a(j):
        slot = jax.lax.rem(j, _NBUF)
        return pltpu.make_async_copy(
            x_hbm.at[pl.ds(j * tile, tile), pl.ds(0, h)],
            xbuf.at[slot, :, pl.ds(0, h)], sems.at[slot])

    def copy_b(j):
        slot = jax.lax.rem(j, _NBUF)
        return pltpu.make_async_copy(
            x_hbm.at[pl.ds(j * tile, tile), pl.ds(h, h)],
            xbuf.at[slot, :, pl.ds(h, h)], sems2.at[slot])

    @pl.when(i == 0)
    def _prologue():
        for j in range(min(_NBUF - 1, nsteps)):
            copy_a(j).start()
            copy_b(j).start()
        wt_ref[...] = jnp.concatenate(
            [wr_ref[...].T, wg_ref[...].T], axis=1)

    @pl.when(i + _NBUF - 1 < nsteps)
    def _lookahead():
        copy_a(i + _NBUF - 1).start()
        copy_b(i + _NBUF - 1).start()

    copy_a(i).wait()
    copy_b(i).wait()
    xt = xbuf[jax.lax.rem(i, _NBUF)]

    y = jnp.dot(xt, wt_ref[...],
                preferred_element_type=jnp.float32)
    yt = y.T
    rlt = yt[:E, :] + br_ref[...].T
    glt = yt[E:, :] + bg_ref[...].T
    rl_ref[...] = rlt
    t = rlt.shape[1]
    iota = jax.lax.broadcasted_iota(jnp.int32, (E, t), 0)

    m1 = jnp.max(rlt, axis=0, keepdims=True)
    i1 = jnp.min(jnp.where(rlt == m1, iota, E), axis=0, keepdims=True)
    masked = jnp.where(iota == i1, -jnp.inf, rlt)
    m2 = jnp.max(masked, axis=0, keepdims=True)
    i2 = jnp.min(jnp.where(masked == m2, iota, E), axis=0, keepdims=True)
    idx_ref[...] = jnp.concatenate([i1, i2], axis=0)

    e2 = jnp.exp(m2 - m1)
    den = 1.0 + e2
    g_ref[...] = jnp.concatenate([1.0 / den, e2 / den], axis=0)

    gm = jnp.max(glt, axis=0, keepdims=True)
    ge = jnp.exp(glt - gm)
    gvt = ge / jnp.sum(ge, axis=0, keepdims=True)
    gv_ref[...] = gvt

    hits = (iota == i1).astype(jnp.float32) + (iota == i2).astype(jnp.float32)
    cnt = jnp.sum(hits, axis=1, keepdims=True)
    imp = jnp.sum(gvt, axis=1, keepdims=True)

    @pl.when(i == 0)
    def _init():
        cnt_ref[...] = jnp.zeros_like(cnt_ref)
        imp_ref[...] = jnp.zeros_like(imp_ref)

    cnt_ref[...] += cnt
    imp_ref[...] += imp

    @pl.when(i == nsteps - 1)
    def _finalize():
        frac = cnt_ref[...] / total_tokens
        bl_ref[...] = (_BALANCE_W
                       * (E * jnp.sum(frac * frac) - 1.0)).reshape(1, 1)
        im = imp_ref[...]
        ti = jnp.sum(im)
        ifrac = jnp.where(ti > 0, im / ti, jnp.zeros_like(im))
        il_ref[...] = (_BALANCE_W
                       * jnp.sum((ifrac - 1.0 / E) ** 2)).reshape(1, 1)


def kernel(x, Wr, br, Wg, bg):
    B, S, D = x.shape
    E = _NUM_EXPERTS
    n_tok = B * S
    tile = 512
    nsteps = n_tok // tile

    xf = x.reshape(n_tok, D)
    br2 = br.reshape(1, E)
    bg2 = bg.reshape(1, E)

    grid_spec = pltpu.PrefetchScalarGridSpec(
        num_scalar_prefetch=0,
        grid=(nsteps,),
        in_specs=[
            pl.BlockSpec(memory_space=pl.ANY),
            pl.BlockSpec((E, D), lambda i: (0, 0)),
            pl.BlockSpec((E, D), lambda i: (0, 0)),
            pl.BlockSpec((1, E), lambda i: (0, 0)),
            pl.BlockSpec((1, E), lambda i: (0, 0)),
        ],
        scratch_shapes=[
            pltpu.MemorySpace.VMEM((_NBUF, tile, D), jnp.float32),
            pltpu.MemorySpace.VMEM((D, 2 * E), jnp.float32),
            pltpu.SemaphoreType.DMA((_NBUF,)),
            pltpu.SemaphoreType.DMA((_NBUF,)),
        ],
        out_specs=[
            pl.BlockSpec((E, tile), lambda i: (0, i)),
            pl.BlockSpec((_TOP_K, tile), lambda i: (0, i)),
            pl.BlockSpec((_TOP_K, tile), lambda i: (0, i)),
            pl.BlockSpec((E, tile), lambda i: (0, i)),
            pl.BlockSpec((E, 1), lambda i: (0, 0)),
            pl.BlockSpec((E, 1), lambda i: (0, 0)),
            pl.BlockSpec((1, 1), lambda i: (0, 0)),
            pl.BlockSpec((1, 1), lambda i: (0, 0)),
        ],
    )

    out_shapes = [
        jax.ShapeDtypeStruct((E, n_tok), jnp.float32),
        jax.ShapeDtypeStruct((_TOP_K, n_tok), jnp.int32),
        jax.ShapeDtypeStruct((_TOP_K, n_tok), jnp.float32),
        jax.ShapeDtypeStruct((E, n_tok), jnp.float32),
        jax.ShapeDtypeStruct((E, 1), jnp.float32),
        jax.ShapeDtypeStruct((E, 1), jnp.float32),
        jax.ShapeDtypeStruct((1, 1), jnp.float32),
        jax.ShapeDtypeStruct((1, 1), jnp.float32),
    ]

    body = functools.partial(_body, nsteps, tile, float(n_tok))
    rl, idx, g, gv, _, _, bl, il = pl.pallas_call(
        body,
        grid_spec=grid_spec,
        out_shape=out_shapes,
    )(xf, Wr, Wg, br2, bg2)

    return (rl.T.reshape(B, S, E),
            idx.T.reshape(B, S, _TOP_K),
            g.T.reshape(B, S, _TOP_K),
            gv.T.reshape(B, S, E),
            bl[0, 0],
            il[0, 0])


# speedup vs baseline: 1.0314x; 1.0050x over previous
import functools

import jax
import jax.numpy as jnp
from jax.experimental import pallas as pl
from jax.experimental.pallas import tpu as pltpu

_D_MODEL = 2048
_NUM_EXPERTS = 16
_TOP_K = 2
_BALANCE_W = 0.01
_NBUF = 8


def _body(nsteps, tile, total_tokens, x_hbm, wr_ref, wg_ref, br_ref, bg_ref,
          rl_ref, idx_ref, g_ref, gv_ref, cnt_ref, imp_ref, bl_ref, il_ref,
          xbuf, wt_ref, sems, sems2):
    i = pl.program_id(0)
    E = _NUM_EXPERTS

    h = _D_MODEL // 2

    def copy_a(j):
        slot = jax.lax.rem(j, _NBUF)
        return pltpu.make_async_copy(
            x_hbm.at[pl.ds(j * tile, tile), pl.ds(0, h)],
            xbuf.at[slot, :, pl.ds(0, h)], sems.at[slot])

    def copy_b(j):
        slot = jax.lax.rem(j, _NBUF)
        return pltpu.make_async_copy(
            x_hbm.at[pl.ds(j * tile, tile), pl.ds(h, h)],
            xbuf.at[slot, :, pl.ds(h, h)], sems2.at[slot])

    @pl.when(i == 0)
    def _prologue():
        for j in range(min(_NBUF - 1, nsteps)):
            copy_a(j).start()
            copy_b(j).start()
        wt_ref[...] = jnp.concatenate(
            [wr_ref[...].T, wg_ref[...].T], axis=1)

    @pl.when(i + _NBUF - 1 < nsteps)
    def _lookahead():
        copy_a(i + _NBUF - 1).start()
        copy_b(i + _NBUF - 1).start()

    copy_a(i).wait()
    copy_b(i).wait()
    xt = xbuf[jax.lax.rem(i, _NBUF)]

    y = jnp.dot(xt, wt_ref[...],
                preferred_element_type=jnp.float32)
    yt = y.T
    rlt = yt[:E, :] + br_ref[...].T
    glt = yt[E:, :] + bg_ref[...].T
    rl_ref[...] = rlt
    t = rlt.shape[1]
    iota = jax.lax.broadcasted_iota(jnp.int32, (E, t), 0)

    m1 = jnp.max(rlt, axis=0, keepdims=True)
    i1 = jnp.min(jnp.where(rlt == m1, iota, E), axis=0, keepdims=True)
    masked = jnp.where(iota == i1, -jnp.inf, rlt)
    m2 = jnp.max(masked, axis=0, keepdims=True)
    i2 = jnp.min(jnp.where(masked == m2, iota, E), axis=0, keepdims=True)
    idx_ref[...] = jnp.concatenate([i1, i2], axis=0)

    e2 = jnp.exp(m2 - m1)
    den = 1.0 + e2
    g_ref[...] = jnp.concatenate([1.0 / den, e2 / den], axis=0)

    gm = jnp.max(glt, axis=0, keepdims=True)
    ge = jnp.exp(glt - gm)
    gvt = ge / jnp.sum(ge, axis=0, keepdims=True)
    gv_ref[...] = gvt

    hits = (iota == i1).astype(jnp.float32) + (iota == i2).astype(jnp.float32)
    cnt = jnp.sum(hits, axis=1, keepdims=True)
    imp = jnp.sum(gvt, axis=1, keepdims=True)

    @pl.when(i == 0)
    def _init():
        cnt_ref[...] = jnp.zeros_like(cnt_ref)
        imp_ref[...] = jnp.zeros_like(imp_ref)

    cnt_ref[...] += cnt
    imp_ref[...] += imp

    @pl.when(i == nsteps - 1)
    def _finalize():
        frac = cnt_ref[...] / total_tokens
        bl_ref[...] = (_BALANCE_W
                       * (E * jnp.sum(frac * frac) - 1.0)).reshape(1, 1)
        im = imp_ref[...]
        ti = jnp.sum(im)
        ifrac = jnp.where(ti > 0, im / ti, jnp.zeros_like(im))
        il_ref[...] = (_BALANCE_W
                       * jnp.sum((ifrac - 1.0 / E) ** 2)).reshape(1, 1)


def kernel(x, Wr, br, Wg, bg):
    B, S, D = x.shape
    E = _NUM_EXPERTS
    n_tok = B * S
    tile = 512
    nsteps = n_tok // tile

    xf = x.reshape(n_tok, D)
    br2 = br.reshape(1, E)
    bg2 = bg.reshape(1, E)

    grid_spec = pltpu.PrefetchScalarGridSpec(
        num_scalar_prefetch=0,
        grid=(nsteps,),
        in_specs=[
            pl.BlockSpec(memory_space=pl.ANY),
            pl.BlockSpec((E, D), lambda i: (0, 0)),
            pl.BlockSpec((E, D), lambda i: (0, 0)),
            pl.BlockSpec((1, E), lambda i: (0, 0)),
            pl.BlockSpec((1, E), lambda i: (0, 0)),
        ],
        scratch_shapes=[
            pltpu.MemorySpace.VMEM((_NBUF, tile, D), jnp.float32),
            pltpu.MemorySpace.VMEM((D, 2 * E), jnp.float32),
            pltpu.SemaphoreType.DMA((_NBUF,)),
            pltpu.SemaphoreType.DMA((_NBUF,)),
        ],
        out_specs=[
            pl.BlockSpec((E, tile), lambda i: (0, i)),
            pl.BlockSpec((_TOP_K, tile), lambda i: (0, i)),
            pl.BlockSpec((_TOP_K, tile), lambda i: (0, i)),
            pl.BlockSpec((E, tile), lambda i: (0, i)),
            pl.BlockSpec((E, 1), lambda i: (0, 0)),
            pl.BlockSpec((E, 1), lambda i: (0, 0)),
            pl.BlockSpec((1, 1), lambda i: (0, 0)),
            pl.BlockSpec((1, 1), lambda i: (0, 0)),
        ],
    )

    out_shapes = [
        jax.ShapeDtypeStruct((E, n_tok), jnp.float32),
        jax.ShapeDtypeStruct((_TOP_K, n_tok), jnp.int32),
        jax.ShapeDtypeStruct((_TOP_K, n_tok), jnp.float32),
        jax.ShapeDtypeStruct((E, n_tok), jnp.float32),
        jax.ShapeDtypeStruct((E, 1), jnp.float32),
        jax.ShapeDtypeStruct((E, 1), jnp.float32),
        jax.ShapeDtypeStruct((1, 1), jnp.float32),
        jax.ShapeDtypeStruct((1, 1), jnp.float32),
    ]

    body = functools.partial(_body, nsteps, tile, float(n_tok))
    rl, idx, g, gv, _, _, bl, il = pl.pallas_call(
        body,
        grid_spec=grid_spec,
        out_shape=out_shapes,
    )(xf, Wr, Wg, br2, bg2)

    return (rl.T.reshape(B, S, E),
            idx.T.reshape(B, S, _TOP_K),
            g.T.reshape(B, S, _TOP_K),
            gv.T.reshape(B, S, E),
            bl[0, 0],
            il[0, 0])
